# baseline (speedup 1.0000x reference)
import functools

import jax
import jax.numpy as jnp
from jax import lax
from jax.experimental import pallas as pl
from jax.experimental.pallas import tpu as pltpu
from jax.experimental.pallas import tpu_sc as plsc

L = 16
CH = 128
CHY = 64


def _sc_broadcast(h, w, d, f, bs):
    n_rows = bs * 3 * h * w * d
    rows_blk = w * d
    mesh = plsc.VectorSubcoreMesh(
        core_axis_name="c", subcore_axis_name="s", num_cores=2
    )

    @functools.partial(
        pl.kernel,
        mesh=mesh,
        out_type=jax.ShapeDtypeStruct((n_rows, f), jnp.float32),
        scratch_types=[
            pltpu.VMEM((h, f), jnp.float32),
            pltpu.VMEM((w, f), jnp.float32),
            pltpu.VMEM((d, f), jnp.float32),
            pltpu.VMEM((CH, f), jnp.float32),
            pltpu.VMEM((CH, f), jnp.float32),
            pltpu.VMEM((2, CHY, f), jnp.float32),
            pltpu.SemaphoreType.DMA,
            pltpu.SemaphoreType.DMA,
            pltpu.SemaphoreType.DMA,
            pltpu.SemaphoreType.DMA,
        ],
    )
    def run(xt_hbm, yt_hbm, zt_hbm, out_hbm, xt_v, yt_v, zt_v,
            pbx, pbz, pby, semx, semy0, semy1, semz):
        wid = lax.axis_index("s") * 2 + lax.axis_index("c")
        pltpu.sync_copy(xt_hbm, xt_v)
        pltpu.sync_copy(yt_hbm, yt_v)
        pltpu.sync_copy(zt_hbm, zt_v)
        semy = [semy0, semy1]

        n_chy = rows_blk // CHY
        jpc = CHY // d
        for c in range(n_chy):
            par = c % 2
            if c >= 2:
                for _ in range(bs):
                    pltpu.make_async_copy(
                        pby.at[par], out_hbm.at[pl.ds(0, CHY)], semy[par]
                    ).wait()

            def fy_body(r, _, c=c, par=par):
                j = c * jpc + lax.shift_right_logical(r, 5)
                for q in range(f // L):
                    pby[par, r, pl.ds(q * L, L)] = yt_v[j, pl.ds(q * L, L)]
                return 0
            lax.fori_loop(0, CHY, fy_body, 0)

            for b in range(bs):
                g = (b * 3 + 1) * h + wid
                row0 = g * rows_blk + c * CHY
                pltpu.async_copy(
                    pby.at[par], out_hbm.at[pl.ds(row0, CHY)], semy[par]
                )

        xv = [xt_v[wid, pl.ds(q * L, L)] for q in range(f // L)]

        def fx_body(r, _):
            for q in range(f // L):
                pbx[r, pl.ds(q * L, L)] = xv[q]
            return 0
        lax.fori_loop(0, CH, fx_body, 0)

        n_chx = rows_blk // CH

        def x_body(m, _):
            b = m // n_chx
            ch = lax.rem(m, n_chx)
            g = b * 3 * h + wid
            row0 = g * rows_blk + ch * CH
            pltpu.async_copy(pbx, out_hbm.at[pl.ds(row0, CH)], semx)
            return 0
        lax.fori_loop(0, bs * n_chx, x_body, 0)

        def fz_body(r, _):
            k = lax.bitwise_and(r, d - 1)
            for q in range(f // L):
                pbz[r, pl.ds(q * L, L)] = zt_v[k, pl.ds(q * L, L)]
            return 0
        lax.fori_loop(0, CH, fz_body, 0)

        def z_body(m, _):
            b = m // n_chx
            ch = lax.rem(m, n_chx)
            g = (b * 3 + 2) * h + wid
            row0 = g * rows_blk + ch * CH
            pltpu.async_copy(pbz, out_hbm.at[pl.ds(row0, CH)], semz)
            return 0
        lax.fori_loop(0, bs * n_chx, z_body, 0)

        for _ in range(bs):
            for par in (0, 1):
                pltpu.make_async_copy(
                    pby.at[par], out_hbm.at[pl.ds(0, CHY)], semy[par]
                ).wait()

        def drain_x(m, _):
            pltpu.make_async_copy(
                pbx, out_hbm.at[pl.ds(0, CH)], semx
            ).wait()
            return 0
        lax.fori_loop(0, bs * n_chx, drain_x, 0)

        def drain_z(m, _):
            pltpu.make_async_copy(
                pbz, out_hbm.at[pl.ds(0, CH)], semz
            ).wait()
            return 0
        lax.fori_loop(0, bs * n_chx, drain_z, 0)

    return run


@jax.jit
def kernel(x, x_table, y_table, z_table):
    bs, _, h, w, d = x.shape
    f = x_table.shape[-1]
    flat = _sc_broadcast(h, w, d, f, bs)(x_table, y_table, z_table)
    return flat.reshape(bs, 3, h, w, d, f)


# speedup vs baseline: 1.2947x; 1.2947x over previous
import functools

import jax
import jax.numpy as jnp
from jax import lax
from jax.experimental import pallas as pl
from jax.experimental.pallas import tpu as pltpu
from jax.experimental.pallas import tpu_sc as plsc

L = 16
CH = 128
CHY = 64


def _sc_broadcast(h, w, d, f, bs):
    n_rows = bs * 3 * h * w * d
    rows_blk = w * d
    mesh = plsc.VectorSubcoreMesh(
        core_axis_name="c", subcore_axis_name="s", num_cores=2
    )

    @functools.partial(
        pl.kernel,
        mesh=mesh,
        out_type=jax.ShapeDtypeStruct((n_rows, f), jnp.float32),
        scratch_types=[
            pltpu.VMEM((h, f), jnp.float32),
            pltpu.VMEM((w, f), jnp.float32),
            pltpu.VMEM((d, f), jnp.float32),
            pltpu.VMEM((CH, f), jnp.float32),
            pltpu.VMEM((CH, f), jnp.float32),
            pltpu.VMEM((2, CHY, f), jnp.float32),
            pltpu.SemaphoreType.DMA,
            pltpu.SemaphoreType.DMA,
            pltpu.SemaphoreType.DMA,
            pltpu.SemaphoreType.DMA,
        ],
    )
    def run(xt_hbm, yt_hbm, zt_hbm, out_hbm, xt_v, yt_v, zt_v,
            pbx, pbz, pby, semx, semy0, semy1, semz):
        wid = lax.axis_index("s") * 2 + lax.axis_index("c")
        pltpu.sync_copy(xt_hbm, xt_v)
        pltpu.sync_copy(yt_hbm, yt_v)
        pltpu.sync_copy(zt_hbm, zt_v)
        semy = [semy0, semy1]

        xv = [xt_v[wid, pl.ds(q * L, L)] for q in range(f // L)]

        def fx_body(r, _):
            for q in range(f // L):
                pbx[r, pl.ds(q * L, L)] = xv[q]
            return 0
        lax.fori_loop(0, CH, fx_body, 0)

        n_chx = rows_blk // CH

        def x_body(m, _):
            b = m // n_chx
            ch = lax.rem(m, n_chx)
            g = b * 3 * h + wid
            row0 = g * rows_blk + ch * CH
            pltpu.async_copy(pbx, out_hbm.at[pl.ds(row0, CH)], semx)
            return 0
        lax.fori_loop(0, bs * n_chx, x_body, 0)

        def fz_body(r, _):
            k = lax.bitwise_and(r, d - 1)
            for q in range(f // L):
                pbz[r, pl.ds(q * L, L)] = zt_v[k, pl.ds(q * L, L)]
            return 0
        lax.fori_loop(0, CH, fz_body, 0)

        def z_body(m, _):
            b = m // n_chx
            ch = lax.rem(m, n_chx)
            g = (b * 3 + 2) * h + wid
            row0 = g * rows_blk + ch * CH
            pltpu.async_copy(pbz, out_hbm.at[pl.ds(row0, CH)], semz)
            return 0
        lax.fori_loop(0, bs * n_chx, z_body, 0)

        n_chy = rows_blk // CHY
        jpc = CHY // d
        for c in range(n_chy):
            par = c % 2
            if c >= 2:
                for _ in range(bs):
                    pltpu.make_async_copy(
                        pby.at[par], out_hbm.at[pl.ds(0, CHY)], semy[par]
                    ).wait()

            def fy_body(r, _, c=c, par=par):
                j = c * jpc + lax.shift_right_logical(r, 5)
                for q in range(f // L):
                    pby[par, r, pl.ds(q * L, L)] = yt_v[j, pl.ds(q * L, L)]
                return 0
            lax.fori_loop(0, CHY, fy_body, 0)

            for b in range(bs):
                g = (b * 3 + 1) * h + wid
                row0 = g * rows_blk + c * CHY
                pltpu.async_copy(
                    pby.at[par], out_hbm.at[pl.ds(row0, CHY)], semy[par]
                )

        for _ in range(bs):
            for par in (0, 1):
                pltpu.make_async_copy(
                    pby.at[par], out_hbm.at[pl.ds(0, CHY)], semy[par]
                ).wait()

        def drain_x(m, _):
            pltpu.make_async_copy(
                pbx, out_hbm.at[pl.ds(0, CH)], semx
            ).wait()
            return 0
        lax.fori_loop(0, bs * n_chx, drain_x, 0)

        def drain_z(m, _):
            pltpu.make_async_copy(
                pbz, out_hbm.at[pl.ds(0, CH)], semz
            ).wait()
            return 0
        lax.fori_loop(0, bs * n_chx, drain_z, 0)

    return run


@jax.jit
def kernel(x, x_table, y_table, z_table):
    bs, _, h, w, d = x.shape
    f = x_table.shape[-1]
    flat = _sc_broadcast(h, w, d, f, bs)(x_table, y_table, z_table)
    return flat.reshape(bs, 3, h, w, d, f)
